# Initial kernel scaffold; baseline (speedup 1.0000x reference)
import jax
import jax.numpy as jnp
from jax.experimental import pallas as pl


def kernel(A, B, C_faulty):
    raise NotImplementedError("write your pallas kernel here")


# speedup vs baseline: 2.1702x; 2.1702x over previous
import jax
import jax.numpy as jnp
from jax.experimental import pallas as pl
from jax.experimental.pallas import tpu as pltpu

_BLK = 256
_ATOL = 1e-3
_RTOL = 1e-4


def _checksum_kernel(a_ref, b_ref, c_ref, out_ref, mask_ref,
                     ac_ref, bc_ref, cca_ref):
    bi = pl.program_id(0)
    nbi = pl.num_programs(0)

    c = c_ref[...]
    out_ref[...] = c

    n = c.shape[1]
    nbj = n // _BLK

    colsum = jnp.sum(c, axis=0, keepdims=True)
    ind = (jax.lax.broadcasted_iota(jnp.int32, (n, nbj), 0) // _BLK
           == jax.lax.broadcasted_iota(jnp.int32, (n, nbj), 1)
           ).astype(jnp.float32)
    bsums = jax.lax.dot_general(
        colsum, ind, (((1,), (0,)), ((), ())),
        preferred_element_type=jnp.float32)

    asum = jnp.sum(a_ref[...], axis=0, keepdims=True)
    bsum = jnp.sum(b_ref[...], axis=0, keepdims=True)

    @pl.when(bi == 0)
    def _init():
        ac_ref[...] = jnp.zeros_like(ac_ref)
        bc_ref[...] = jnp.zeros_like(bc_ref)
        cca_ref[...] = jnp.zeros_like(cca_ref)

    rows = jax.lax.broadcasted_iota(jnp.int32, (nbi, 1), 0)
    onehot = (rows == bi).astype(jnp.float32)
    ac_ref[...] += onehot * asum
    bc_ref[...] += onehot * bsum
    cca_ref[...] += onehot * bsums

    @pl.when(bi == nbi - 1)
    def _finish():
        ccc = jax.lax.dot_general(
            bc_ref[...], ac_ref[...], (((1,), (1,)), ((), ())),
            preferred_element_type=jnp.float32)
        cca = cca_ref[...]
        mask_ref[...] = (jnp.abs(cca - ccc)
                         > _ATOL + _RTOL * jnp.abs(ccc)).astype(jnp.int32)


def _correct_kernel(idx_ref, b_ref, a_ref, c_any_ref, out_ref):
    del idx_ref, c_any_ref
    out_ref[...] = jax.lax.dot_general(
        b_ref[...], a_ref[...], (((1,), (1,)), ((), ())),
        preferred_element_type=jnp.float32,
        precision=jax.lax.Precision.HIGHEST)


def kernel(A, B, C_faulty):
    m, n = C_faulty.shape
    kin = A.shape[1]
    nbi = m // _BLK
    nbj = n // _BLK

    c_through, mask = pl.pallas_call(
        _checksum_kernel,
        grid=(nbi,),
        in_specs=[
            pl.BlockSpec((_BLK, kin), lambda i: (i, 0)),
            pl.BlockSpec((_BLK, kin), lambda i: (i, 0)),
            pl.BlockSpec((_BLK, n), lambda i: (i, 0)),
        ],
        out_specs=[
            pl.BlockSpec((_BLK, n), lambda i: (i, 0)),
            pl.BlockSpec((nbi, nbj), lambda i: (0, 0)),
        ],
        out_shape=[
            jax.ShapeDtypeStruct((m, n), jnp.float32),
            jax.ShapeDtypeStruct((nbi, nbj), jnp.int32),
        ],
        scratch_shapes=[
            pltpu.VMEM((nbi, kin), jnp.float32),
            pltpu.VMEM((nbi, kin), jnp.float32),
            pltpu.VMEM((nbi, nbj), jnp.float32),
        ],
        compiler_params=pltpu.CompilerParams(
            dimension_semantics=("arbitrary",)),
    )(A, B, C_faulty)

    flat = mask.reshape(-1)
    count = jnp.sum(flat)
    num = jnp.maximum(count, 1)
    idx = jnp.nonzero(flat, size=flat.shape[0], fill_value=0)[0].astype(
        jnp.int32)

    grid_spec = pltpu.PrefetchScalarGridSpec(
        num_scalar_prefetch=1,
        grid=(num,),
        in_specs=[
            pl.BlockSpec((_BLK, kin), lambda s, idx: (idx[s] // nbj, 0)),
            pl.BlockSpec((_BLK, kin), lambda s, idx: (idx[s] % nbj, 0)),
            pl.BlockSpec(memory_space=pl.ANY),
        ],
        out_specs=pl.BlockSpec(
            (_BLK, _BLK), lambda s, idx: (idx[s] // nbj, idx[s] % nbj)),
    )
    corrected = pl.pallas_call(
        _correct_kernel,
        grid_spec=grid_spec,
        out_shape=jax.ShapeDtypeStruct((m, n), jnp.float32),
        input_output_aliases={3: 0},
        compiler_params=pltpu.CompilerParams(
            dimension_semantics=("arbitrary",)),
    )(idx, B, A, c_through)
    return corrected


# speedup vs baseline: 2.2717x; 1.0468x over previous
import jax
import jax.numpy as jnp
from jax.experimental import pallas as pl
from jax.experimental.pallas import tpu as pltpu

_BLK = 256
_ATOL = 1e-3
_RTOL = 1e-4


def _checksum_kernel(a_ref, b_ref, c_ref, out_ref, idx_ref, cnt_ref,
                     ac_ref, bc_ref, cca_ref):
    bi = pl.program_id(0)
    nbi = pl.num_programs(0)

    c = c_ref[...]
    out_ref[...] = c

    n = c.shape[1]
    nbj = n // _BLK

    colsum = jnp.sum(c, axis=0, keepdims=True)
    ind = (jax.lax.broadcasted_iota(jnp.int32, (n, nbj), 0) // _BLK
           == jax.lax.broadcasted_iota(jnp.int32, (n, nbj), 1)
           ).astype(jnp.float32)
    bsums = jax.lax.dot_general(
        colsum, ind, (((1,), (0,)), ((), ())),
        preferred_element_type=jnp.float32)

    asum = jnp.sum(a_ref[...], axis=0, keepdims=True)
    bsum = jnp.sum(b_ref[...], axis=0, keepdims=True)

    @pl.when(bi == 0)
    def _init():
        ac_ref[...] = jnp.zeros_like(ac_ref)
        bc_ref[...] = jnp.zeros_like(bc_ref)
        cca_ref[...] = jnp.zeros_like(cca_ref)

    rows = jax.lax.broadcasted_iota(jnp.int32, (nbi, 1), 0)
    onehot = (rows == bi).astype(jnp.float32)
    ac_ref[...] += onehot * asum
    bc_ref[...] += onehot * bsum
    cca_ref[...] += onehot * bsums

    @pl.when(bi == nbi - 1)
    def _finish():
        ccc = jax.lax.dot_general(
            bc_ref[...], ac_ref[...], (((1,), (1,)), ((), ())),
            preferred_element_type=jnp.float32)
        cca = cca_ref[...]
        mf = (jnp.abs(cca - ccc)
              > _ATOL + _RTOL * jnp.abs(ccc)).astype(jnp.float32)

        ltu = (jax.lax.broadcasted_iota(jnp.int32, (nbj, nbj), 0)
               <= jax.lax.broadcasted_iota(jnp.int32, (nbj, nbj), 1)
               ).astype(jnp.float32)
        inrow = jax.lax.dot_general(
            mf, ltu, (((1,), (0,)), ((), ())),
            preferred_element_type=jnp.float32)
        rowtot = inrow[:, nbj - 1:nbj]
        lts = (jax.lax.broadcasted_iota(jnp.int32, (nbi, nbi), 1)
               < jax.lax.broadcasted_iota(jnp.int32, (nbi, nbi), 0)
               ).astype(jnp.float32)
        rowpref = jax.lax.dot_general(
            lts, rowtot, (((1,), (0,)), ((), ())),
            preferred_element_type=jnp.float32)
        rank = ((rowpref + inrow) * mf).astype(jnp.int32)

        nslots = nbi * nbj
        s3 = jax.lax.broadcasted_iota(jnp.int32, (nslots, nbi, nbj), 0)
        sel = (rank[None] == s3 + 1).astype(jnp.int32)
        fidx3 = (jax.lax.broadcasted_iota(jnp.int32, (nslots, nbi, nbj), 1)
                 * nbj
                 + jax.lax.broadcasted_iota(jnp.int32, (nslots, nbi, nbj), 2))
        idx_ref[...] = jnp.sum(jnp.sum(sel * fidx3, axis=2), axis=1,
                               keepdims=True)
        cnt_ref[...] = jnp.sum(mf).astype(jnp.int32).reshape(1, 1)


def _correct_kernel(idx_ref, b_ref, a_ref, c_any_ref, out_ref):
    del idx_ref, c_any_ref
    out_ref[...] = jax.lax.dot_general(
        b_ref[...], a_ref[...], (((1,), (1,)), ((), ())),
        preferred_element_type=jnp.float32,
        precision=jax.lax.Precision.HIGHEST)


def kernel(A, B, C_faulty):
    m, n = C_faulty.shape
    kin = A.shape[1]
    nbi = m // _BLK
    nbj = n // _BLK
    nslots = nbi * nbj

    c_through, idx2, cnt = pl.pallas_call(
        _checksum_kernel,
        grid=(nbi,),
        in_specs=[
            pl.BlockSpec((_BLK, kin), lambda i: (i, 0)),
            pl.BlockSpec((_BLK, kin), lambda i: (i, 0)),
            pl.BlockSpec((_BLK, n), lambda i: (i, 0)),
        ],
        out_specs=[
            pl.BlockSpec((_BLK, n), lambda i: (i, 0)),
            pl.BlockSpec((nslots, 1), lambda i: (0, 0)),
            pl.BlockSpec((1, 1), lambda i: (0, 0)),
        ],
        out_shape=[
            jax.ShapeDtypeStruct((m, n), jnp.float32),
            jax.ShapeDtypeStruct((nslots, 1), jnp.int32),
            jax.ShapeDtypeStruct((1, 1), jnp.int32),
        ],
        scratch_shapes=[
            pltpu.VMEM((nbi, kin), jnp.float32),
            pltpu.VMEM((nbi, kin), jnp.float32),
            pltpu.VMEM((nbi, nbj), jnp.float32),
        ],
        compiler_params=pltpu.CompilerParams(
            dimension_semantics=("arbitrary",)),
    )(A, B, C_faulty)

    num = jnp.maximum(cnt[0, 0], 1)
    idx = idx2.reshape(-1)

    grid_spec = pltpu.PrefetchScalarGridSpec(
        num_scalar_prefetch=1,
        grid=(num,),
        in_specs=[
            pl.BlockSpec((_BLK, kin), lambda s, idx: (idx[s] // nbj, 0)),
            pl.BlockSpec((_BLK, kin), lambda s, idx: (idx[s] % nbj, 0)),
            pl.BlockSpec(memory_space=pl.ANY),
        ],
        out_specs=pl.BlockSpec(
            (_BLK, _BLK), lambda s, idx: (idx[s] // nbj, idx[s] % nbj)),
    )
    corrected = pl.pallas_call(
        _correct_kernel,
        grid_spec=grid_spec,
        out_shape=jax.ShapeDtypeStruct((m, n), jnp.float32),
        input_output_aliases={3: 0},
        compiler_params=pltpu.CompilerParams(
            dimension_semantics=("arbitrary",)),
    )(idx, B, A, c_through)
    return corrected
